# baseline (speedup 1.0000x reference)
import functools

import jax
import jax.numpy as jnp
from jax import lax
from jax.experimental import pallas as pl
from jax.experimental.pallas import tpu as pltpu
from jax.experimental.pallas import tpu_sc as plsc

B = 4096
L = 200
EMB = 128
NUM_EMO = 6
L_PAD = 224
HALF = L_PAD // 2
NW = 32
S_PER_W = B // NW
NVH = HALF // 16


def _emotion_kernel_body(ids_hbm, table_hbm, wb_hbm, out_hbm, ids_v, rows_v, wb_v, out_v, sem):
    wid = lax.axis_index("s") * 2 + lax.axis_index("c")
    base = wid * S_PER_W
    pltpu.sync_copy(ids_hbm.at[pl.ds(base, S_PER_W)], ids_v)
    pltpu.sync_copy(wb_hbm, wb_v)

    def sentence(s, carry):
        cp0 = pltpu.async_copy(
            table_hbm.at[ids_v.at[s, 0]], rows_v.at[pl.ds(0, HALF)], sem)
        cp1 = pltpu.async_copy(
            table_hbm.at[ids_v.at[s, 1]], rows_v.at[pl.ds(HALF, HALF)], sem)

        cnt = jnp.zeros((16,), jnp.int32)
        for c in range(2):
            for j in range(NVH):
                tok = ids_v[s, c, pl.ds(j * 16, 16)]
                cnt = cnt + plsc.all_reduce_population_count(tok != 0)
        inv = 1.0 / jnp.maximum(cnt.astype(jnp.float32), 1e-9)

        cp0.wait()
        cp1.wait()

        def row_add(t, accs):
            return tuple(a + rows_v[t, pl.ds(k * 16, 16)] for k, a in enumerate(accs))
        accs = lax.fori_loop(0, L_PAD, row_add,
                             tuple(jnp.zeros((16,), jnp.float32) for _ in range(8)))

        out_v[s, 0, :] = wb_v[NUM_EMO, pl.ds(0, 16)]
        out_row = out_v.at[s, 0]
        for e in range(NUM_EMO):
            dot = jnp.zeros((16,), jnp.float32)
            for k in range(8):
                dot = dot + accs[k] * wb_v[e, pl.ds(k * 16, 16)]
            plsc.addupdate_scatter(out_row, [jnp.full((16,), e, jnp.int32)], dot * inv)
        return carry

    lax.fori_loop(0, S_PER_W, sentence, 0)
    pltpu.sync_copy(out_v, out_hbm.at[pl.ds(base, S_PER_W)])


@functools.cache
def _build():
    return pl.kernel(
        _emotion_kernel_body,
        mesh=plsc.VectorSubcoreMesh(core_axis_name="c", subcore_axis_name="s"),
        compiler_params=pltpu.CompilerParams(needs_layout_passes=False),
        out_type=jax.ShapeDtypeStruct((B, 1, 16), jnp.float32),
        scratch_types=[
            pltpu.VMEM((S_PER_W, 2, HALF), jnp.int32),
            pltpu.VMEM((L_PAD, EMB), jnp.float32),
            pltpu.VMEM((8, EMB), jnp.float32),
            pltpu.VMEM((S_PER_W, 1, 16), jnp.float32),
            pltpu.SemaphoreType.DMA,
        ],
    )


def kernel(input_ids, table, W_emo, b_emo):
    ids_pad = jnp.pad(input_ids, ((0, 0), (0, L_PAD - L))).reshape(B, 2, HALF)
    wb = jnp.zeros((8, EMB), jnp.float32)
    wb = wb.at[:NUM_EMO, :].set(W_emo.T).at[NUM_EMO, :NUM_EMO].set(b_emo)
    out = _build()(ids_pad, table, wb)
    return out.reshape(B, 16)[:, :NUM_EMO]


# speedup vs baseline: 1.0023x; 1.0023x over previous
import functools

import jax
import jax.numpy as jnp
from jax import lax
from jax.experimental import pallas as pl
from jax.experimental.pallas import tpu as pltpu
from jax.experimental.pallas import tpu_sc as plsc

B = 4096
L = 200
EMB = 128
NUM_EMO = 6
L_PAD = 224
HALF = L_PAD // 2
NW = 32
S_PER_W = B // NW
NVH = HALF // 16
UNROLL = 4


def _emotion_kernel_body(ids_hbm, table_hbm, wb_hbm, out_hbm,
                         ids_v, rows_a, rows_b, wb_v, out_v, sem_a, sem_b):
    wid = lax.axis_index("s") * 2 + lax.axis_index("c")
    base = wid * S_PER_W
    pltpu.sync_copy(ids_hbm.at[pl.ds(base, S_PER_W)], ids_v)
    pltpu.sync_copy(wb_hbm, wb_v)

    def issue(s, rows, sem):
        pltpu.async_copy(table_hbm.at[ids_v.at[s, 0]], rows.at[pl.ds(0, HALF)], sem)
        pltpu.async_copy(table_hbm.at[ids_v.at[s, 1]], rows.at[pl.ds(HALF, HALF)], sem)

    issue(0, rows_a, sem_a)
    issue(1, rows_b, sem_b)

    def pair(i, carry):
        for b, (rows, sem) in enumerate(((rows_a, sem_a), (rows_b, sem_b))):
            s = 2 * i + b

            cnt = jnp.zeros((16,), jnp.int32)
            for c in range(2):
                for j in range(NVH):
                    tok = ids_v[s, c, pl.ds(j * 16, 16)]
                    cnt = cnt + plsc.all_reduce_population_count(tok != 0)
            inv = 1.0 / jnp.maximum(cnt.astype(jnp.float32), 1e-9)

            pltpu.make_async_copy(table_hbm.at[pl.ds(0, L_PAD)], rows, sem).wait()

            def row_add(t, accs):
                accs = list(accs)
                for u in range(UNROLL):
                    for k in range(8):
                        accs[k] = accs[k] + rows[t * UNROLL + u, pl.ds(k * 16, 16)]
                return tuple(accs)
            accs = lax.fori_loop(0, L_PAD // UNROLL, row_add,
                                 tuple(jnp.zeros((16,), jnp.float32) for _ in range(8)))

            out_v[s, 0, :] = wb_v[NUM_EMO, pl.ds(0, 16)]
            out_row = out_v.at[s, 0]
            for e in range(NUM_EMO):
                dot = jnp.zeros((16,), jnp.float32)
                for k in range(8):
                    dot = dot + accs[k] * wb_v[e, pl.ds(k * 16, 16)]
                plsc.addupdate_scatter(out_row, [jnp.full((16,), e, jnp.int32)],
                                       dot * inv)

            @pl.when(i < S_PER_W // 2 - 1)
            def _():
                issue(s + 2, rows, sem)
        return carry

    lax.fori_loop(0, S_PER_W // 2, pair, 0)
    pltpu.sync_copy(out_v, out_hbm.at[pl.ds(base, S_PER_W)])


@functools.cache
def _build():
    return pl.kernel(
        _emotion_kernel_body,
        mesh=plsc.VectorSubcoreMesh(core_axis_name="c", subcore_axis_name="s"),
        compiler_params=pltpu.CompilerParams(needs_layout_passes=False),
        out_type=jax.ShapeDtypeStruct((B, 1, 16), jnp.float32),
        scratch_types=[
            pltpu.VMEM((S_PER_W, 2, HALF), jnp.int32),
            pltpu.VMEM((L_PAD, EMB), jnp.float32),
            pltpu.VMEM((L_PAD, EMB), jnp.float32),
            pltpu.VMEM((8, EMB), jnp.float32),
            pltpu.VMEM((S_PER_W, 1, 16), jnp.float32),
            pltpu.SemaphoreType.DMA,
            pltpu.SemaphoreType.DMA,
        ],
    )


def kernel(input_ids, table, W_emo, b_emo):
    ids_pad = jnp.pad(input_ids, ((0, 0), (0, L_PAD - L))).reshape(B, 2, HALF)
    wb = jnp.zeros((8, EMB), jnp.float32)
    wb = wb.at[:NUM_EMO, :].set(W_emo.T).at[NUM_EMO, :NUM_EMO].set(b_emo)
    out = _build()(ids_pad, table, wb)
    return out.reshape(B, 16)[:, :NUM_EMO]


# speedup vs baseline: 22.1649x; 22.1138x over previous
import functools

import jax
import jax.numpy as jnp
from jax import lax
from jax.experimental import pallas as pl
from jax.experimental.pallas import tpu as pltpu
from jax.experimental.pallas import tpu_sc as plsc

B = 4096
L = 200
EMB = 128
NUM_EMO = 6
HALF = L // 2
NW = 32
S_PER_W = B // NW
NBUF = 3
UNROLL = 4


def _emotion_kernel_body(ids_hbm, table_hbm, wb_hbm, out_hbm,
                         ids_v, rows_a, rows_b, rows_c, wb_v, out_v,
                         sem_a, sem_b, sem_c):
    wid = lax.axis_index("s") * 2 + lax.axis_index("c")
    base = wid * S_PER_W
    pltpu.sync_copy(ids_hbm.at[pl.ds(base, S_PER_W)], ids_v)
    pltpu.sync_copy(wb_hbm, wb_v)

    bufs = ((rows_a, sem_a), (rows_b, sem_b), (rows_c, sem_c))

    def issue(s, rows, sem):
        pltpu.async_copy(table_hbm.at[ids_v.at[s, 0]], rows.at[pl.ds(0, HALF)], sem)
        pltpu.async_copy(table_hbm.at[ids_v.at[s, 1]], rows.at[pl.ds(HALF, HALF)], sem)

    for b in range(NBUF):
        issue(b, bufs[b][0], bufs[b][1])

    lane = lax.iota(jnp.int32, 16)

    def group(i, carry):
        for b, (rows, sem) in enumerate(bufs):
            s = NBUF * i + b

            cnt = jnp.zeros((16,), jnp.int32)
            for c in range(2):
                for j in range(6):
                    tok = ids_v[s, c, pl.ds(j * 16, 16)]
                    cnt = cnt + plsc.all_reduce_population_count(tok != 0)
                tok = ids_v[s, c, pl.ds(HALF - 16, 16)]
                cnt = cnt + plsc.all_reduce_population_count(
                    (tok != 0) & (lane >= 12))
            inv = 1.0 / jnp.maximum(cnt.astype(jnp.float32), 1e-9)

            pltpu.make_async_copy(table_hbm.at[pl.ds(0, L)], rows, sem).wait()

            def row_add(t, accs):
                accs = list(accs)
                for u in range(UNROLL):
                    for k in range(8):
                        accs[k] = accs[k] + rows[t * UNROLL + u, pl.ds(k * 16, 16)]
                return tuple(accs)
            accs = lax.fori_loop(0, L // UNROLL, row_add,
                                 tuple(jnp.zeros((16,), jnp.float32) for _ in range(8)))

            out_v[s, 0, :] = wb_v[NUM_EMO, pl.ds(0, 16)]
            out_row = out_v.at[s, 0]
            for e in range(NUM_EMO):
                dot = jnp.zeros((16,), jnp.float32)
                for k in range(8):
                    dot = dot + accs[k] * wb_v[e, pl.ds(k * 16, 16)]
                plsc.addupdate_scatter(out_row, [jnp.full((16,), e, jnp.int32)],
                                       dot * inv)

            @pl.when(s + NBUF < S_PER_W)
            def _():
                issue(s + NBUF, rows, sem)
        return carry

    lax.fori_loop(0, S_PER_W // NBUF, group, 0)
    for b in range(S_PER_W % NBUF):
        s = (S_PER_W // NBUF) * NBUF + b
        rows, sem = bufs[b]
        cnt = jnp.zeros((16,), jnp.int32)
        for c in range(2):
            for j in range(6):
                tok = ids_v[s, c, pl.ds(j * 16, 16)]
                cnt = cnt + plsc.all_reduce_population_count(tok != 0)
            tok = ids_v[s, c, pl.ds(HALF - 16, 16)]
            cnt = cnt + plsc.all_reduce_population_count((tok != 0) & (lane >= 12))
        inv = 1.0 / jnp.maximum(cnt.astype(jnp.float32), 1e-9)
        pltpu.make_async_copy(table_hbm.at[pl.ds(0, L)], rows, sem).wait()

        def row_add(t, accs):
            accs = list(accs)
            for u in range(UNROLL):
                for k in range(8):
                    accs[k] = accs[k] + rows[t * UNROLL + u, pl.ds(k * 16, 16)]
            return tuple(accs)
        accs = lax.fori_loop(0, L // UNROLL, row_add,
                             tuple(jnp.zeros((16,), jnp.float32) for _ in range(8)))
        out_v[s, 0, :] = wb_v[NUM_EMO, pl.ds(0, 16)]
        out_row = out_v.at[s, 0]
        for e in range(NUM_EMO):
            dot = jnp.zeros((16,), jnp.float32)
            for k in range(8):
                dot = dot + accs[k] * wb_v[e, pl.ds(k * 16, 16)]
            plsc.addupdate_scatter(out_row, [jnp.full((16,), e, jnp.int32)],
                                   dot * inv)

    pltpu.sync_copy(out_v, out_hbm.at[pl.ds(base, S_PER_W)])


@functools.cache
def _build():
    return pl.kernel(
        _emotion_kernel_body,
        mesh=plsc.VectorSubcoreMesh(core_axis_name="c", subcore_axis_name="s"),
        compiler_params=pltpu.CompilerParams(needs_layout_passes=False),
        out_type=jax.ShapeDtypeStruct((B, 1, 16), jnp.float32),
        scratch_types=[
            pltpu.VMEM((S_PER_W, 2, HALF), jnp.int32),
            pltpu.VMEM((L, EMB), jnp.float32),
            pltpu.VMEM((L, EMB), jnp.float32),
            pltpu.VMEM((L, EMB), jnp.float32),
            pltpu.VMEM((8, EMB), jnp.float32),
            pltpu.VMEM((S_PER_W, 1, 16), jnp.float32),
            pltpu.SemaphoreType.DMA,
            pltpu.SemaphoreType.DMA,
            pltpu.SemaphoreType.DMA,
        ],
    )


def kernel(input_ids, table, W_emo, b_emo):
    ids2 = input_ids.reshape(B, 2, HALF)
    wb = jnp.zeros((8, EMB), jnp.float32)
    wb = wb.at[:NUM_EMO, :].set(W_emo.T).at[NUM_EMO, :NUM_EMO].set(b_emo)
    out = _build()(ids2, table, wb)
    return out.reshape(B, 16)[:, :NUM_EMO]


# speedup vs baseline: 22.2741x; 1.0049x over previous
import functools

import jax
import jax.numpy as jnp
from jax import lax
from jax.experimental import pallas as pl
from jax.experimental.pallas import tpu as pltpu
from jax.experimental.pallas import tpu_sc as plsc

B = 4096
L = 200
EMB = 128
NUM_EMO = 6
HALF = L // 2
NW = 32
S_PER_W = B // NW
NBUF = 3
UNROLL = 8


def _emotion_kernel_body(ids_hbm, table_hbm, wb_hbm, out_hbm,
                         ids_v, rows_a, rows_b, rows_c, wb_v, out_v,
                         sem_a, sem_b, sem_c):
    wid = lax.axis_index("s") * 2 + lax.axis_index("c")
    base = wid * S_PER_W
    pltpu.sync_copy(ids_hbm.at[pl.ds(base, S_PER_W)], ids_v)
    pltpu.sync_copy(wb_hbm, wb_v)

    bufs = ((rows_a, sem_a), (rows_b, sem_b), (rows_c, sem_c))

    def issue(s, rows, sem):
        pltpu.async_copy(table_hbm.at[ids_v.at[s, 0]], rows.at[pl.ds(0, HALF)], sem)
        pltpu.async_copy(table_hbm.at[ids_v.at[s, 1]], rows.at[pl.ds(HALF, HALF)], sem)

    for b in range(NBUF):
        issue(b, bufs[b][0], bufs[b][1])

    lane = lax.iota(jnp.int32, 16)

    def group(i, carry):
        for b, (rows, sem) in enumerate(bufs):
            s = NBUF * i + b

            cnt0 = jnp.zeros((16,), jnp.int32)
            cnt1 = jnp.zeros((16,), jnp.int32)
            for c in range(2):
                for j in range(3):
                    cnt0 = cnt0 + (ids_v[s, c, pl.ds(j * 16, 16)] != 0)
                for j in range(3, 6):
                    cnt1 = cnt1 + (ids_v[s, c, pl.ds(j * 16, 16)] != 0)
                tok = ids_v[s, c, pl.ds(HALF - 16, 16)]
                cnt0 = cnt0 + ((tok != 0) & (lane >= 12))
            cnt = jnp.broadcast_to(jnp.sum(cnt0 + cnt1), (16,))
            inv = 1.0 / jnp.maximum(cnt.astype(jnp.float32), 1e-9)

            pltpu.make_async_copy(table_hbm.at[pl.ds(0, L)], rows, sem).wait()

            def row_add(t, accs):
                accs = list(accs)
                for u in range(UNROLL):
                    for k in range(8):
                        accs[k] = accs[k] + rows[t * UNROLL + u, pl.ds(k * 16, 16)]
                return tuple(accs)
            accs = lax.fori_loop(0, L // UNROLL, row_add,
                                 tuple(jnp.zeros((16,), jnp.float32) for _ in range(8)))

            out_v[s, 0, :] = wb_v[NUM_EMO, pl.ds(0, 16)]
            out_row = out_v.at[s, 0]
            for e in range(NUM_EMO):
                d0 = jnp.zeros((16,), jnp.float32)
                d1 = jnp.zeros((16,), jnp.float32)
                for k in range(4):
                    d0 = d0 + accs[k] * wb_v[e, pl.ds(k * 16, 16)]
                for k in range(4, 8):
                    d1 = d1 + accs[k] * wb_v[e, pl.ds(k * 16, 16)]
                dot = d0 + d1
                plsc.addupdate_scatter(out_row, [jnp.full((16,), e, jnp.int32)],
                                       dot * inv)

            @pl.when(s + NBUF < S_PER_W)
            def _():
                issue(s + NBUF, rows, sem)
        return carry

    lax.fori_loop(0, S_PER_W // NBUF, group, 0)
    for b in range(S_PER_W % NBUF):
        s = (S_PER_W // NBUF) * NBUF + b
        rows, sem = bufs[b]
        cnt0 = jnp.zeros((16,), jnp.int32)
        cnt1 = jnp.zeros((16,), jnp.int32)
        for c in range(2):
            for j in range(3):
                cnt0 = cnt0 + (ids_v[s, c, pl.ds(j * 16, 16)] != 0)
            for j in range(3, 6):
                cnt1 = cnt1 + (ids_v[s, c, pl.ds(j * 16, 16)] != 0)
            tok = ids_v[s, c, pl.ds(HALF - 16, 16)]
            cnt0 = cnt0 + ((tok != 0) & (lane >= 12))
        cnt = jnp.broadcast_to(jnp.sum(cnt0 + cnt1), (16,))
        inv = 1.0 / jnp.maximum(cnt.astype(jnp.float32), 1e-9)
        pltpu.make_async_copy(table_hbm.at[pl.ds(0, L)], rows, sem).wait()

        def row_add(t, accs):
            accs = list(accs)
            for u in range(UNROLL):
                for k in range(8):
                    accs[k] = accs[k] + rows[t * UNROLL + u, pl.ds(k * 16, 16)]
            return tuple(accs)
        accs = lax.fori_loop(0, L // UNROLL, row_add,
                             tuple(jnp.zeros((16,), jnp.float32) for _ in range(8)))
        out_v[s, 0, :] = wb_v[NUM_EMO, pl.ds(0, 16)]
        out_row = out_v.at[s, 0]
        for e in range(NUM_EMO):
            d0 = jnp.zeros((16,), jnp.float32)
            d1 = jnp.zeros((16,), jnp.float32)
            for k in range(4):
                d0 = d0 + accs[k] * wb_v[e, pl.ds(k * 16, 16)]
            for k in range(4, 8):
                d1 = d1 + accs[k] * wb_v[e, pl.ds(k * 16, 16)]
            dot = d0 + d1
            plsc.addupdate_scatter(out_row, [jnp.full((16,), e, jnp.int32)],
                                   dot * inv)

    pltpu.sync_copy(out_v, out_hbm.at[pl.ds(base, S_PER_W)])


@functools.cache
def _build():
    return pl.kernel(
        _emotion_kernel_body,
        mesh=plsc.VectorSubcoreMesh(core_axis_name="c", subcore_axis_name="s"),
        compiler_params=pltpu.CompilerParams(needs_layout_passes=False),
        out_type=jax.ShapeDtypeStruct((B, 1, 16), jnp.float32),
        scratch_types=[
            pltpu.VMEM((S_PER_W, 2, HALF), jnp.int32),
            pltpu.VMEM((L, EMB), jnp.float32),
            pltpu.VMEM((L, EMB), jnp.float32),
            pltpu.VMEM((L, EMB), jnp.float32),
            pltpu.VMEM((8, EMB), jnp.float32),
            pltpu.VMEM((S_PER_W, 1, 16), jnp.float32),
            pltpu.SemaphoreType.DMA,
            pltpu.SemaphoreType.DMA,
            pltpu.SemaphoreType.DMA,
        ],
    )


def kernel(input_ids, table, W_emo, b_emo):
    ids2 = input_ids.reshape(B, 2, HALF)
    wb = jnp.zeros((8, EMB), jnp.float32)
    wb = wb.at[:NUM_EMO, :].set(W_emo.T).at[NUM_EMO, :NUM_EMO].set(b_emo)
    out = _build()(ids2, table, wb)
    return out.reshape(B, 16)[:, :NUM_EMO]
